# Initial kernel scaffold; baseline (speedup 1.0000x reference)
import jax
import jax.numpy as jnp
from jax.experimental import pallas as pl


def kernel(h, p, e, edge_index, Q_w, Q_b, K_w, K_b, E_w, E_b, V_w, V_b):
    raise NotImplementedError("write your pallas kernel here")


# speedup vs baseline: 20.5857x; 20.5857x over previous
import functools

import jax
import jax.numpy as jnp
from jax import lax
from jax.experimental import pallas as pl
from jax.experimental.pallas import tpu as pltpu
from jax.experimental.pallas import tpu_sc as plsc

_N = 10000
_E = 320000
_H = 8
_D = 16
_DOUT = _H * _D

_C = 40
_NW = 32
_EPW = _E // _NW
_NCH = _EPW // _C
_ZB = 10000
_NAZ = 11264
_NT = _NAZ // 16

_F32 = jnp.float32
_HP = jax.lax.Precision.HIGHEST


def _qkv_call(hc, W, b):
    n = hc.shape[0]
    bm = 400

    def kern(hc_ref, w_ref, b_ref, q_ref, k_ref, v_ref):
        acc = jnp.dot(hc_ref[...], w_ref[...],
                      preferred_element_type=_F32, precision=_HP) + b_ref[...]
        q_ref[...] = acc[:, 0:128]
        k_ref[...] = acc[:, 128:256]
        v_ref[...] = acc[:, 256:384]

    return pl.pallas_call(
        kern,
        grid=(n // bm,),
        in_specs=[pl.BlockSpec((bm, 256), lambda i: (i, 0)),
                  pl.BlockSpec((256, 384), lambda i: (0, 0)),
                  pl.BlockSpec((1, 384), lambda i: (0, 0))],
        out_specs=[pl.BlockSpec((bm, 128), lambda i: (i, 0))] * 3,
        out_shape=[jax.ShapeDtypeStruct((n, 128), _F32)] * 3,
    )(hc, W, b)


def _proj_call(e, W, b):
    m = e.shape[0]
    bm = 1280

    def kern(e_ref, w_ref, b_ref, o_ref):
        o_ref[...] = jnp.dot(e_ref[...], w_ref[...],
                             preferred_element_type=_F32, precision=_HP) + b_ref[...]

    return pl.pallas_call(
        kern,
        grid=(m // bm,),
        in_specs=[pl.BlockSpec((bm, 128), lambda i: (i, 0)),
                  pl.BlockSpec((128, 128), lambda i: (0, 0)),
                  pl.BlockSpec((1, 128), lambda i: (0, 0))],
        out_specs=pl.BlockSpec((bm, 128), lambda i: (i, 0)),
        out_shape=jax.ShapeDtypeStruct((m, 128), _F32),
    )(e, W, b)


def _combine_call(wvp, zp, S):
    bm = 400

    def kern(wv_ref, z_ref, s_ref, o_ref):
        wv = wv_ref[0] + wv_ref[1]
        z = z_ref[0] + z_ref[1]
        zr = jnp.dot(z, s_ref[...], preferred_element_type=_F32, precision=_HP)
        o_ref[...] = wv / (zr + 1e-6)

    return pl.pallas_call(
        kern,
        grid=(_N // bm,),
        in_specs=[pl.BlockSpec((2, bm, 128), lambda i: (0, i, 0)),
                  pl.BlockSpec((2, bm, 8), lambda i: (0, i, 0)),
                  pl.BlockSpec((8, 128), lambda i: (0, 0))],
        out_specs=pl.BlockSpec((bm, 128), lambda i: (i, 0)),
        out_shape=jax.ShapeDtypeStruct((_N, 128), _F32),
    )(wvp, zp, S)


def _sc_edge(qh, kh, vh, ee, src, dst, zer):
    mesh = plsc.VectorSubcoreMesh(core_axis_name="c", subcore_axis_name="s")

    @functools.partial(
        pl.kernel, mesh=mesh,
        out_type=[jax.ShapeDtypeStruct((2, _NAZ, 128), _F32),
                  jax.ShapeDtypeStruct((_E * 16,), _F32)],
        scratch_types=[
            pltpu.VMEM((_C,), jnp.int32),
            pltpu.VMEM((_C,), jnp.int32),
            pltpu.VMEM((_C + 16,), jnp.int32),
            pltpu.VMEM((_C,), jnp.int32),
            pltpu.VMEM((_C, 128), _F32),
            pltpu.VMEM((_C, 128), _F32),
            pltpu.VMEM((_C, 128), _F32),
            pltpu.VMEM((_C, 128), _F32),
            pltpu.VMEM((_C, 128), _F32),
            pltpu.VMEM((_C * 16,), _F32),
            pltpu.VMEM_SHARED((_NAZ, 128), _F32),
            pltpu.SemaphoreType.DMA,
        ],
    )
    def k(qh_h, kh_h, vh_h, ee_h, src_h, dst_h, zer_h, wv_o, attn_o,
          src_v, dst_v, dstp_v, zidx_v, k_v, q_v, v_v, e_v, zr_v, attn_v,
          acc, sem):
        cid = lax.axis_index("c")
        sid = lax.axis_index("s")
        wid = sid * 2 + cid
        r0 = sid * _NT

        pltpu.sync_copy(zer_h.at[pl.ds(r0, _NT)], acc.at[pl.ds(r0, _NT)])
        plsc.subcore_barrier()

        lanes = lax.iota(jnp.int32, 16)
        zvec = jnp.zeros((16,), _F32)
        hmasks = [lanes == hd for hd in range(_H)]
        xor_idx = [lanes ^ k for k in (8, 4, 2, 1)]
        gdn = lax.GatherDimensionNumbers(
            offset_dims=(), collapsed_slice_dims=(0,), start_index_map=(0,))

        def _allsum(x):
            for xi in xor_idx:
                x = x + lax.gather(x, xi[:, None], gdn, (1,),
                                   mode=lax.GatherScatterMode.PROMISE_IN_BOUNDS)
            return x

        def edge_body(ei, carry):
            attn_vec = zvec
            for hd in range(_H):
                sl = pl.ds(hd * 16, 16)
                w = k_v[ei, sl] * q_v[ei, sl] * e_v[ei, sl]
                sv = jnp.exp(jnp.clip(_allsum(w), -5.0, 5.0))
                v_v[ei, sl] = v_v[ei, sl] * sv
                attn_vec = jnp.where(hmasks[hd], sv, attn_vec)
            attn_v[pl.ds(ei * 16, 16)] = attn_vec
            for t in range(8):
                zr_v[ei, pl.ds(t * 16, 16)] = zvec
            zoff = (dstp_v[pl.ds(ei, 16)][0] & 7) * 16
            zr_v[ei, pl.ds(zoff, 16)] = attn_vec
            return carry

        def chunk_body(j, carry):
            base = wid * _EPW + j * _C
            pltpu.sync_copy(src_h.at[pl.ds(base, _C)], src_v)
            pltpu.sync_copy(dst_h.at[pl.ds(base, _C)], dst_v)
            pltpu.sync_copy(dst_h.at[pl.ds(base, _C)], dstp_v.at[pl.ds(0, _C)])
            c1 = pltpu.async_copy(kh_h.at[src_v], k_v, sem)
            c2 = pltpu.async_copy(qh_h.at[dst_v], q_v, sem)
            c3 = pltpu.async_copy(vh_h.at[src_v], v_v, sem)
            pltpu.sync_copy(ee_h.at[pl.ds(base, _C)], e_v)
            for (lo, so) in ((0, 0), (16, 16), (24, 24)):
                d = dstp_v[pl.ds(lo, 16)]
                zidx_v[pl.ds(so, 16)] = _ZB + lax.shift_right_logical(d, 3)
            c1.wait()
            c2.wait()
            c3.wait()
            lax.fori_loop(0, _C, edge_body, 0)
            pltpu.sync_copy(v_v, acc.at[dst_v], add=True)
            pltpu.sync_copy(zr_v, acc.at[zidx_v], add=True)
            pltpu.sync_copy(attn_v, attn_o.at[pl.ds(base * 16, _C * 16)])
            return carry

        lax.fori_loop(0, _NCH, chunk_body, 0)
        plsc.subcore_barrier()
        pltpu.sync_copy(acc.at[pl.ds(r0, _NT)], wv_o.at[cid, pl.ds(r0, _NT)])

    return k(qh, kh, vh, ee, src, dst, zer)


def kernel(h, p, e, edge_index, Q_w, Q_b, K_w, K_b, E_w, E_b, V_w, V_b):
    hc = jnp.concatenate([h, p], axis=1)
    W = jnp.concatenate([Q_w, K_w, V_w], axis=1)
    b = jnp.concatenate([Q_b, K_b, V_b])[None, :]
    qh, kh, vh = _qkv_call(hc, W, b)
    ee = _proj_call(e, E_w * 0.25, E_b[None, :] * 0.25)
    src = edge_index[0]
    dst = edge_index[1]
    zer = jnp.zeros((_NAZ, 128), _F32)
    accd, attn_flat = _sc_edge(qh, kh, vh, ee, src, dst, zer)
    S = jnp.repeat(jnp.eye(_H, dtype=_F32), _D, axis=1)
    zp = accd[:, _ZB:_ZB + _N // 8, :].reshape(2, _N, 16)[:, :, 0:8]
    h_out = _combine_call(accd[:, :_N, :], zp, S)
    return (h_out.reshape(_N, _H, _D),
            attn_flat.reshape(_E, 16)[:, 0:8].reshape(_E, _H, 1))


# speedup vs baseline: 23.4521x; 1.1392x over previous
import functools

import jax
import jax.numpy as jnp
from jax import lax
from jax.experimental import pallas as pl
from jax.experimental.pallas import tpu as pltpu
from jax.experimental.pallas import tpu_sc as plsc

_N = 10000
_E = 320000
_H = 8
_D = 16
_DOUT = _H * _D

_C = 32
_NW = 32
_TCH = _E // _C
_JFULL = _TCH // _NW
_NPAIR = _JFULL // 2
_ZB = 10000
_NAZ = 10752
_NT = _NAZ // 16

_F32 = jnp.float32
_HP = jax.lax.Precision.HIGHEST


def _qkv_call(hc, W, b):
    n = hc.shape[0]
    bm = 400

    def kern(hc_ref, w_ref, b_ref, q_ref, k_ref, v_ref):
        acc = jnp.dot(hc_ref[...], w_ref[...],
                      preferred_element_type=_F32, precision=_HP) + b_ref[...]
        q_ref[...] = acc[:, 0:128]
        k_ref[...] = acc[:, 128:256]
        v_ref[...] = acc[:, 256:384]

    return pl.pallas_call(
        kern,
        grid=(n // bm,),
        in_specs=[pl.BlockSpec((bm, 256), lambda i: (i, 0)),
                  pl.BlockSpec((256, 384), lambda i: (0, 0)),
                  pl.BlockSpec((1, 384), lambda i: (0, 0))],
        out_specs=[pl.BlockSpec((bm, 128), lambda i: (i, 0))] * 3,
        out_shape=[jax.ShapeDtypeStruct((n, 128), _F32)] * 3,
    )(hc, W, b)


def _proj_call(e, W, b):
    m = e.shape[0]
    bm = 1280

    def kern(e_ref, w_ref, b_ref, o_ref):
        o_ref[...] = jnp.dot(e_ref[...], w_ref[...],
                             preferred_element_type=_F32, precision=_HP) + b_ref[...]

    return pl.pallas_call(
        kern,
        grid=(m // bm,),
        in_specs=[pl.BlockSpec((bm, 128), lambda i: (i, 0)),
                  pl.BlockSpec((128, 128), lambda i: (0, 0)),
                  pl.BlockSpec((1, 128), lambda i: (0, 0))],
        out_specs=pl.BlockSpec((bm, 128), lambda i: (i, 0)),
        out_shape=jax.ShapeDtypeStruct((m, 128), _F32),
    )(e, W, b)


def _combine_call(wvp, zp, S):
    bm = 400

    def kern(wv_ref, z_ref, s_ref, o_ref):
        wv = wv_ref[0] + wv_ref[1]
        z = z_ref[0] + z_ref[1]
        zr = jnp.dot(z, s_ref[...], preferred_element_type=_F32, precision=_HP)
        o_ref[...] = wv / (zr + 1e-6)

    return pl.pallas_call(
        kern,
        grid=(_N // bm,),
        in_specs=[pl.BlockSpec((2, bm, 128), lambda i: (0, i, 0)),
                  pl.BlockSpec((2, bm, 8), lambda i: (0, i, 0)),
                  pl.BlockSpec((8, 128), lambda i: (0, 0))],
        out_specs=pl.BlockSpec((bm, 128), lambda i: (i, 0)),
        out_shape=jax.ShapeDtypeStruct((_N, 128), _F32),
    )(wvp, zp, S)


def _sc_edge(qh, kh, vh, ee, src, dst, zer):
    mesh = plsc.VectorSubcoreMesh(core_axis_name="c", subcore_axis_name="s")

    def _bufset():
        return [
            pltpu.VMEM((_C,), jnp.int32),
            pltpu.VMEM((_C,), jnp.int32),
            pltpu.VMEM((_C + 16,), jnp.int32),
            pltpu.VMEM((_C,), jnp.int32),
            pltpu.VMEM((_C,), jnp.int32),
            pltpu.VMEM((_C, 128), _F32),
            pltpu.VMEM((_C, 128), _F32),
            pltpu.VMEM((_C, 128), _F32),
            pltpu.VMEM((_C, 128), _F32),
            pltpu.VMEM((_C, 128), _F32),
            pltpu.VMEM((_C * 16,), _F32),
            pltpu.SemaphoreType.DMA,
            pltpu.SemaphoreType.DMA,
            pltpu.SemaphoreType.DMA,
        ]

    @functools.partial(
        pl.kernel, mesh=mesh,
        out_type=[jax.ShapeDtypeStruct((2, _NAZ, 128), _F32),
                  jax.ShapeDtypeStruct((_E * 16,), _F32)],
        scratch_types=(_bufset() + _bufset()
                       + [pltpu.VMEM_SHARED((_NAZ, 128), _F32)]),
    )
    def k(qh_h, kh_h, vh_h, ee_h, src_h, dst_h, zer_h, wv_o, attn_o, *sc):
        bufs = (sc[0:14], sc[14:28])
        acc = sc[28]
        cid = lax.axis_index("c")
        sid = lax.axis_index("s")
        wid = sid * 2 + cid
        r0 = sid * _NT

        pltpu.sync_copy(zer_h.at[pl.ds(r0, _NT)], acc.at[pl.ds(r0, _NT)])
        plsc.subcore_barrier()

        lanes = lax.iota(jnp.int32, 16)
        zvec = jnp.zeros((16,), _F32)
        hmasks = [lanes == hd for hd in range(_H)]
        xor_idx = [lanes ^ kk for kk in (8, 4, 2, 1)]
        rot8_idx = (lanes + 8) & 15
        gdn = lax.GatherDimensionNumbers(
            offset_dims=(), collapsed_slice_dims=(0,), start_index_map=(0,))

        def _shuf(x, idx):
            return lax.gather(x, idx[:, None], gdn, (1,),
                              mode=lax.GatherScatterMode.PROMISE_IN_BOUNDS)

        def _allsum(x):
            for xi in xor_idx:
                x = x + _shuf(x, xi)
            return x

        def base_of(j):
            return (wid + 32 * j) * _C

        def issue_idx(j, B):
            srcv, dstv, dstpv = bufs[B][0], bufs[B][1], bufs[B][2]
            isem = bufs[B][11]
            base = base_of(j)
            pltpu.async_copy(src_h.at[pl.ds(base, _C)], srcv, isem)
            pltpu.async_copy(dst_h.at[pl.ds(base, _C)], dstv, isem)
            pltpu.async_copy(dst_h.at[pl.ds(base, _C)],
                             dstpv.at[pl.ds(0, _C)], isem)

        def wait_idx(j, B):
            srcv, dstv, dstpv = bufs[B][0], bufs[B][1], bufs[B][2]
            isem = bufs[B][11]
            base = base_of(j)
            pltpu.make_async_copy(src_h.at[pl.ds(base, _C)], srcv, isem).wait()
            pltpu.make_async_copy(dst_h.at[pl.ds(base, _C)], dstv, isem).wait()
            pltpu.make_async_copy(dst_h.at[pl.ds(base, _C)],
                                  dstpv.at[pl.ds(0, _C)], isem).wait()

        def issue_gathers(j, B):
            srcv, dstv = bufs[B][0], bufs[B][1]
            kv, qv, vv, ev = bufs[B][5:9]
            gsem = bufs[B][12]
            base = base_of(j)
            pltpu.async_copy(kh_h.at[srcv], kv, gsem)
            pltpu.async_copy(qh_h.at[dstv], qv, gsem)
            pltpu.async_copy(vh_h.at[srcv], vv, gsem)
            pltpu.async_copy(ee_h.at[pl.ds(base, _C)], ev, gsem)

        def wait_gathers(j, B):
            srcv, dstv = bufs[B][0], bufs[B][1]
            kv, qv, vv, ev = bufs[B][5:9]
            gsem = bufs[B][12]
            base = base_of(j)
            pltpu.make_async_copy(kh_h.at[srcv], kv, gsem).wait()
            pltpu.make_async_copy(qh_h.at[dstv], qv, gsem).wait()
            pltpu.make_async_copy(vh_h.at[srcv], vv, gsem).wait()
            pltpu.make_async_copy(ee_h.at[pl.ds(base, _C)], ev, gsem).wait()

        def do_scatters(j, B):
            zidxv, dstv2 = bufs[B][3], bufs[B][4]
            vv, zrv, attnv = bufs[B][7], bufs[B][9], bufs[B][10]
            ssem = bufs[B][13]
            base = base_of(j)
            pltpu.sync_copy(vv, acc.at[dstv2], add=True)
            pltpu.sync_copy(zrv, acc.at[zidxv], add=True)
            c = pltpu.async_copy(attnv, attn_o.at[pl.ds(base * 16, _C * 16)],
                                 ssem)
            c.wait()

        def compute(B):
            dstpv, zidxv, dstv2 = bufs[B][2], bufs[B][3], bufs[B][4]
            kv, qv, vv, ev, zrv, attnv = bufs[B][5:11]
            for o in (0, 16):
                d = dstpv[pl.ds(o, 16)]
                zidxv[pl.ds(o, 16)] = _ZB + lax.shift_right_logical(d, 4)
                dstv2[pl.ds(o, 16)] = d

            def edge_body(ei, carry):
                attn_vec = zvec
                for hd in range(_H):
                    sl = pl.ds(hd * 16, 16)
                    w = kv[ei, sl] * qv[ei, sl] * ev[ei, sl]
                    sv = jnp.exp(jnp.clip(_allsum(w), -5.0, 5.0))
                    vv[ei, sl] = vv[ei, sl] * sv
                    attn_vec = jnp.where(hmasks[hd], sv, attn_vec)
                attnv[pl.ds(ei * 16, 16)] = attn_vec
                d0 = dstpv[pl.ds(ei, 16)][0]
                off = (d0 & 15) * 8
                offc = jnp.minimum(off, 112)
                didx = (lanes - lax.broadcast_in_dim(off - offc, (16,), ())) & 15
                store_vec = _shuf(attn_vec, didx)
                for t in range(8):
                    zrv[ei, pl.ds(t * 16, 16)] = zvec
                zrv[ei, pl.ds(offc, 16)] = store_vec
                return carry

            lax.fori_loop(0, _C, edge_body, 0)

        def pair_body(jp, carry):
            j0 = 2 * jp
            issue_idx(j0 + 1, 1)
            wait_gathers(j0, 0)
            compute(0)
            do_scatters(j0, 0)
            wait_idx(j0 + 1, 1)
            issue_gathers(j0 + 1, 1)

            j1 = j0 + 1

            @pl.when(jp < _NPAIR - 1)
            def _():
                issue_idx(j1 + 1, 0)
            wait_gathers(j1, 1)
            compute(1)
            do_scatters(j1, 1)

            @pl.when(jp < _NPAIR - 1)
            def _():
                wait_idx(j1 + 1, 0)
                issue_gathers(j1 + 1, 0)
            return carry

        issue_idx(0, 0)
        wait_idx(0, 0)
        issue_gathers(0, 0)
        lax.fori_loop(0, _NPAIR, pair_body, 0)

        @pl.when(wid < _TCH - 32 * _JFULL)
        def _():
            jt = _JFULL
            issue_idx(jt, 0)
            wait_idx(jt, 0)
            issue_gathers(jt, 0)
            wait_gathers(jt, 0)
            compute(0)
            do_scatters(jt, 0)

        plsc.subcore_barrier()
        pltpu.sync_copy(acc.at[pl.ds(r0, _NT)], wv_o.at[cid, pl.ds(r0, _NT)])

    return k(qh, kh, vh, ee, src, dst, zer)


def kernel(h, p, e, edge_index, Q_w, Q_b, K_w, K_b, E_w, E_b, V_w, V_b):
    hc = jnp.concatenate([h, p], axis=1)
    W = jnp.concatenate([Q_w, K_w, V_w], axis=1)
    b = jnp.concatenate([Q_b, K_b, V_b])[None, :]
    qh, kh, vh = _qkv_call(hc, W, b)
    ee = _proj_call(e, E_w * 0.25, E_b[None, :] * 0.25)
    src = edge_index[0]
    dst = edge_index[1]
    zer = jnp.zeros((_NAZ, 128), _F32)
    accd, attn_flat = _sc_edge(qh, kh, vh, ee, src, dst, zer)
    S = jnp.repeat(jnp.eye(_H, dtype=_F32), _D, axis=1)
    zp = accd[:, _ZB:_ZB + _N // 16, :].reshape(2, _N, 8)
    h_out = _combine_call(accd[:, :_N, :], zp, S)
    return (h_out.reshape(_N, _H, _D),
            attn_flat.reshape(_E, 16)[:, 0:8].reshape(_E, _H, 1))


# speedup vs baseline: 24.3860x; 1.0398x over previous
import functools

import jax
import jax.numpy as jnp
from jax import lax
from jax.experimental import pallas as pl
from jax.experimental.pallas import tpu as pltpu
from jax.experimental.pallas import tpu_sc as plsc

_N = 10000
_E = 320000
_H = 8
_D = 16
_DOUT = _H * _D

_C = 32
_NW = 32
_TCH = _E // _C
_JFULL = _TCH // _NW
_NPAIR = _JFULL // 2
_ZB = 10000
_NAZ = 10752
_NT = _NAZ // 16

_F32 = jnp.float32
_HP = jax.lax.Precision.HIGHEST


def _qkv_call(hc, W, b):
    n = hc.shape[0]
    bm = 400

    def kern(hc_ref, w_ref, b_ref, q_ref, k_ref, v_ref):
        acc = jnp.dot(hc_ref[...], w_ref[...],
                      preferred_element_type=_F32) + b_ref[...]
        q_ref[...] = acc[:, 0:128]
        k_ref[...] = acc[:, 128:256]
        v_ref[...] = acc[:, 256:384]

    return pl.pallas_call(
        kern,
        grid=(n // bm,),
        in_specs=[pl.BlockSpec((bm, 256), lambda i: (i, 0)),
                  pl.BlockSpec((256, 384), lambda i: (0, 0)),
                  pl.BlockSpec((1, 384), lambda i: (0, 0))],
        out_specs=[pl.BlockSpec((bm, 128), lambda i: (i, 0))] * 3,
        out_shape=[jax.ShapeDtypeStruct((n, 128), _F32)] * 3,
    )(hc, W, b)


def _proj_call(e, W, b):
    m = e.shape[0]
    bm = 1280

    def kern(e_ref, w_ref, b_ref, o_ref):
        o_ref[...] = jnp.dot(e_ref[...], w_ref[...],
                             preferred_element_type=_F32) + b_ref[...]

    return pl.pallas_call(
        kern,
        grid=(m // bm,),
        in_specs=[pl.BlockSpec((bm, 128), lambda i: (i, 0)),
                  pl.BlockSpec((128, 128), lambda i: (0, 0)),
                  pl.BlockSpec((1, 128), lambda i: (0, 0))],
        out_specs=pl.BlockSpec((bm, 128), lambda i: (i, 0)),
        out_shape=jax.ShapeDtypeStruct((m, 128), _F32),
    )(e, W, b)


def _combine_call(wvp, zp, S):
    bm = 400

    def kern(wv_ref, z_ref, s_ref, o_ref):
        wv = wv_ref[0] + wv_ref[1]
        z = z_ref[0] + z_ref[1]
        zr = jnp.dot(z, s_ref[...], preferred_element_type=_F32, precision=_HP)
        o_ref[...] = wv / (zr + 1e-6)

    return pl.pallas_call(
        kern,
        grid=(_N // bm,),
        in_specs=[pl.BlockSpec((2, bm, 128), lambda i: (0, i, 0)),
                  pl.BlockSpec((2, bm, 8), lambda i: (0, i + _ZB * 16 // bm, 0)),
                  pl.BlockSpec((8, 128), lambda i: (0, 0))],
        out_specs=pl.BlockSpec((bm, 128), lambda i: (i, 0)),
        out_shape=jax.ShapeDtypeStruct((_N, 128), _F32),
    )(wvp, zp, S)


def _sc_edge(qh, kh, vh, ee, src, dst, zer):
    mesh = plsc.VectorSubcoreMesh(core_axis_name="c", subcore_axis_name="s")

    def _bufset():
        return [
            pltpu.VMEM((_C,), jnp.int32),
            pltpu.VMEM((_C,), jnp.int32),
            pltpu.VMEM((_C + 16,), jnp.int32),
            pltpu.VMEM((_C,), jnp.int32),
            pltpu.VMEM((_C,), jnp.int32),
            pltpu.VMEM((_C, 128), _F32),
            pltpu.VMEM((_C, 128), _F32),
            pltpu.VMEM((_C, 128), _F32),
            pltpu.VMEM((_C, 128), _F32),
            pltpu.VMEM((_C, 128), _F32),
            pltpu.VMEM((_C * 8,), _F32),
            pltpu.SemaphoreType.DMA,
            pltpu.SemaphoreType.DMA,
            pltpu.SemaphoreType.DMA,
        ]

    @functools.partial(
        pl.kernel, mesh=mesh,
        out_type=[jax.ShapeDtypeStruct((2, _NAZ, 128), _F32),
                  jax.ShapeDtypeStruct((_E * 8,), _F32)],
        scratch_types=(_bufset() + _bufset()
                       + [pltpu.VMEM_SHARED((_NAZ, 128), _F32)]),
    )
    def k(qh_h, kh_h, vh_h, ee_h, src_h, dst_h, zer_h, wv_o, attn_o, *sc):
        bufs = (sc[0:14], sc[14:28])
        acc = sc[28]
        cid = lax.axis_index("c")
        sid = lax.axis_index("s")
        wid = sid * 2 + cid
        r0 = sid * _NT

        pltpu.sync_copy(zer_h.at[pl.ds(r0, _NT)], acc.at[pl.ds(r0, _NT)])
        plsc.subcore_barrier()

        lanes = lax.iota(jnp.int32, 16)
        zvec = jnp.zeros((16,), _F32)
        hmasks = [lanes == hd for hd in range(_H)]
        xor_idx = [lanes ^ kk for kk in (8, 4, 2, 1)]
        rot8_idx = (lanes + 8) & 15
        gdn = lax.GatherDimensionNumbers(
            offset_dims=(), collapsed_slice_dims=(0,), start_index_map=(0,))

        def _shuf(x, idx):
            return lax.gather(x, idx[:, None], gdn, (1,),
                              mode=lax.GatherScatterMode.PROMISE_IN_BOUNDS)

        def _allsum(x):
            for xi in xor_idx:
                x = x + _shuf(x, xi)
            return x

        def base_of(j):
            return (wid + 32 * j) * _C

        def issue_idx(j, B):
            srcv, dstv, dstpv = bufs[B][0], bufs[B][1], bufs[B][2]
            isem = bufs[B][11]
            base = base_of(j)
            pltpu.async_copy(src_h.at[pl.ds(base, _C)], srcv, isem)
            pltpu.async_copy(dst_h.at[pl.ds(base, _C)], dstv, isem)
            pltpu.async_copy(dst_h.at[pl.ds(base, _C)],
                             dstpv.at[pl.ds(0, _C)], isem)

        def wait_idx(j, B):
            srcv, dstv, dstpv = bufs[B][0], bufs[B][1], bufs[B][2]
            isem = bufs[B][11]
            base = base_of(j)
            pltpu.make_async_copy(src_h.at[pl.ds(base, _C)], srcv, isem).wait()
            pltpu.make_async_copy(dst_h.at[pl.ds(base, _C)], dstv, isem).wait()
            pltpu.make_async_copy(dst_h.at[pl.ds(base, _C)],
                                  dstpv.at[pl.ds(0, _C)], isem).wait()

        def issue_gathers(j, B):
            srcv, dstv = bufs[B][0], bufs[B][1]
            kv, qv, vv, ev = bufs[B][5:9]
            gsem = bufs[B][12]
            base = base_of(j)
            pltpu.async_copy(kh_h.at[srcv], kv, gsem)
            pltpu.async_copy(qh_h.at[dstv], qv, gsem)
            pltpu.async_copy(vh_h.at[srcv], vv, gsem)
            pltpu.async_copy(ee_h.at[pl.ds(base, _C)], ev, gsem)

        def wait_gathers(j, B):
            srcv, dstv = bufs[B][0], bufs[B][1]
            kv, qv, vv, ev = bufs[B][5:9]
            gsem = bufs[B][12]
            base = base_of(j)
            pltpu.make_async_copy(kh_h.at[srcv], kv, gsem).wait()
            pltpu.make_async_copy(qh_h.at[dstv], qv, gsem).wait()
            pltpu.make_async_copy(vh_h.at[srcv], vv, gsem).wait()
            pltpu.make_async_copy(ee_h.at[pl.ds(base, _C)], ev, gsem).wait()

        def do_scatters(j, B):
            zidxv, dstv2 = bufs[B][3], bufs[B][4]
            vv, zrv, attnv = bufs[B][7], bufs[B][9], bufs[B][10]
            ssem = bufs[B][13]
            base = base_of(j)
            pltpu.sync_copy(vv, acc.at[dstv2], add=True)
            pltpu.sync_copy(zrv, acc.at[zidxv], add=True)
            c = pltpu.async_copy(attnv, attn_o.at[pl.ds(base * 8, _C * 8)],
                                 ssem)
            c.wait()

        def compute(B):
            dstpv, zidxv, dstv2 = bufs[B][2], bufs[B][3], bufs[B][4]
            kv, qv, vv, ev, zrv, attnv = bufs[B][5:11]
            for o in (0, 16):
                d = dstpv[pl.ds(o, 16)]
                zidxv[pl.ds(o, 16)] = _ZB + lax.shift_right_logical(d, 4)
                dstv2[pl.ds(o, 16)] = d

            def edge_body(ei, carry):
                attn_vec = zvec
                for hd in range(_H):
                    sl = pl.ds(hd * 16, 16)
                    w = kv[ei, sl] * qv[ei, sl] * ev[ei, sl]
                    sv = jnp.exp(jnp.clip(_allsum(w), -5.0, 5.0))
                    vv[ei, sl] = vv[ei, sl] * sv
                    attn_vec = jnp.where(hmasks[hd], sv, attn_vec)
                parity = ei & 1
                combined = carry + _shuf(attn_vec, rot8_idx)

                @pl.when(parity == 1)
                def _():
                    attnv[pl.ds((ei >> 1) * 16, 16)] = combined
                pf = 1.0 - lax.convert_element_type(parity, _F32)
                new_carry = attn_vec * lax.broadcast_in_dim(pf, (16,), ())
                d0 = dstpv[pl.ds(ei, 16)][0]
                off = (d0 & 15) * 8
                offc = jnp.minimum(off, 112)
                didx = (lanes - lax.broadcast_in_dim(off - offc, (16,), ())) & 15
                store_vec = _shuf(attn_vec, didx)
                for t in range(8):
                    zrv[ei, pl.ds(t * 16, 16)] = zvec
                zrv[ei, pl.ds(offc, 16)] = store_vec
                return new_carry

            lax.fori_loop(0, _C, edge_body, zvec)

        def pair_body(jp, carry):
            j0 = 2 * jp
            issue_idx(j0 + 1, 1)
            wait_gathers(j0, 0)
            compute(0)
            do_scatters(j0, 0)
            wait_idx(j0 + 1, 1)
            issue_gathers(j0 + 1, 1)

            j1 = j0 + 1

            @pl.when(jp < _NPAIR - 1)
            def _():
                issue_idx(j1 + 1, 0)
            wait_gathers(j1, 1)
            compute(1)
            do_scatters(j1, 1)

            @pl.when(jp < _NPAIR - 1)
            def _():
                wait_idx(j1 + 1, 0)
                issue_gathers(j1 + 1, 0)
            return carry

        issue_idx(0, 0)
        wait_idx(0, 0)
        issue_gathers(0, 0)
        lax.fori_loop(0, _NPAIR, pair_body, 0)

        @pl.when(wid < _TCH - 32 * _JFULL)
        def _():
            jt = _JFULL
            issue_idx(jt, 0)
            wait_idx(jt, 0)
            issue_gathers(jt, 0)
            wait_gathers(jt, 0)
            compute(0)
            do_scatters(jt, 0)

        plsc.subcore_barrier()
        pltpu.sync_copy(acc.at[pl.ds(r0, _NT)], wv_o.at[cid, pl.ds(r0, _NT)])

    return k(qh, kh, vh, ee, src, dst, zer)


def kernel(h, p, e, edge_index, Q_w, Q_b, K_w, K_b, E_w, E_b, V_w, V_b):
    hc = jnp.concatenate([h, p], axis=1)
    W = jnp.concatenate([Q_w, K_w, V_w], axis=1)
    b = jnp.concatenate([Q_b, K_b, V_b])[None, :]
    qh, kh, vh = _qkv_call(hc, W, b)
    ee = _proj_call(e, E_w * 0.25, E_b[None, :] * 0.25)
    src = edge_index[0]
    dst = edge_index[1]
    zer = jnp.zeros((_NAZ, 128), _F32)
    accd, attn_flat = _sc_edge(qh, kh, vh, ee, src, dst, zer)
    S = jnp.repeat(jnp.eye(_H, dtype=_F32), _D, axis=1)
    h_out = _combine_call(accd, accd.reshape(2, _NAZ * 16, 8), S)
    return (h_out.reshape(_N, _H, _D), attn_flat.reshape(_E, _H, 1))


# speedup vs baseline: 24.6361x; 1.0103x over previous
import functools

import jax
import jax.numpy as jnp
from jax import lax
from jax.experimental import pallas as pl
from jax.experimental.pallas import tpu as pltpu
from jax.experimental.pallas import tpu_sc as plsc

_N = 10000
_E = 320000
_H = 8
_D = 16
_DOUT = _H * _D

_C = 32
_NW = 32
_TCH = _E // _C
_JFULL = _TCH // _NW
_NPAIR = _JFULL // 2
_ZB = 10000
_NAZ = 10752
_NT = _NAZ // 16

_F32 = jnp.float32
_HP = jax.lax.Precision.HIGHEST


def _qkv_call(h, p, W1, W2, b):
    n = h.shape[0]
    bm = 400

    def kern(h_ref, p_ref, w1_ref, w2_ref, b_ref, q_ref, k_ref, v_ref):
        acc = (jnp.dot(h_ref[...], w1_ref[...], preferred_element_type=_F32)
               + jnp.dot(p_ref[...], w2_ref[...], preferred_element_type=_F32)
               + b_ref[...])
        q_ref[...] = acc[:, 0:128]
        k_ref[...] = acc[:, 128:256]
        v_ref[...] = acc[:, 256:384]

    return pl.pallas_call(
        kern,
        grid=(n // bm,),
        in_specs=[pl.BlockSpec((bm, 128), lambda i: (i, 0)),
                  pl.BlockSpec((bm, 128), lambda i: (i, 0)),
                  pl.BlockSpec((128, 384), lambda i: (0, 0)),
                  pl.BlockSpec((128, 384), lambda i: (0, 0)),
                  pl.BlockSpec((1, 384), lambda i: (0, 0))],
        out_specs=[pl.BlockSpec((bm, 128), lambda i: (i, 0))] * 3,
        out_shape=[jax.ShapeDtypeStruct((n, 128), _F32)] * 3,
    )(h, p, W1, W2, b)


def _proj_call(e, W, b):
    m = e.shape[0]
    bm = 1280

    def kern(e_ref, w_ref, b_ref, o_ref):
        o_ref[...] = jnp.dot(e_ref[...], w_ref[...],
                             preferred_element_type=_F32) + b_ref[...]

    return pl.pallas_call(
        kern,
        grid=(m // bm,),
        in_specs=[pl.BlockSpec((bm, 128), lambda i: (i, 0)),
                  pl.BlockSpec((128, 128), lambda i: (0, 0)),
                  pl.BlockSpec((1, 128), lambda i: (0, 0))],
        out_specs=pl.BlockSpec((bm, 128), lambda i: (i, 0)),
        out_shape=jax.ShapeDtypeStruct((m, 128), _F32),
    )(e, W, b)


def _combine_call(wvp, zp, S):
    bm = 400

    def kern(wv_ref, z_ref, s_ref, o_ref):
        wv = wv_ref[0] + wv_ref[1]
        z = z_ref[0] + z_ref[1]
        zr = jnp.dot(z, s_ref[...], preferred_element_type=_F32, precision=_HP)
        o_ref[...] = wv / (zr + 1e-6)

    return pl.pallas_call(
        kern,
        grid=(_N // bm,),
        in_specs=[pl.BlockSpec((2, bm, 128), lambda i: (0, i, 0)),
                  pl.BlockSpec((2, bm, 8), lambda i: (0, i + _ZB * 16 // bm, 0)),
                  pl.BlockSpec((8, 128), lambda i: (0, 0))],
        out_specs=pl.BlockSpec((bm, 128), lambda i: (i, 0)),
        out_shape=jax.ShapeDtypeStruct((_N, 128), _F32),
    )(wvp, zp, S)


def _sc_edge(qh, kh, vh, ee, src, dst, zer):
    mesh = plsc.VectorSubcoreMesh(core_axis_name="c", subcore_axis_name="s")

    def _bufset():
        return [
            pltpu.VMEM((_C,), jnp.int32),
            pltpu.VMEM((_C,), jnp.int32),
            pltpu.VMEM((_C + 16,), jnp.int32),
            pltpu.VMEM((2 * _C,), jnp.int32),
            pltpu.VMEM((_C, 128), _F32),
            pltpu.VMEM((_C, 128), _F32),
            pltpu.VMEM((_C, 128), _F32),
            pltpu.VMEM((2 * _C, 128), _F32),
            pltpu.VMEM((_C * 8,), _F32),
            pltpu.SemaphoreType.DMA,
            pltpu.SemaphoreType.DMA,
            pltpu.SemaphoreType.DMA,
        ]

    @functools.partial(
        pl.kernel, mesh=mesh,
        out_type=[jax.ShapeDtypeStruct((2, _NAZ, 128), _F32),
                  jax.ShapeDtypeStruct((_E * 8,), _F32)],
        scratch_types=(_bufset() + _bufset()
                       + [pltpu.VMEM_SHARED((_NAZ, 128), _F32)]),
    )
    def k(qh_h, kh_h, vh_h, ee_h, src_h, dst_h, zer_h, wv_o, attn_o, *sc):
        bufs = (sc[0:12], sc[12:24])
        acc = sc[24]
        cid = lax.axis_index("c")
        sid = lax.axis_index("s")
        wid = sid * 2 + cid
        r0 = sid * _NT

        pltpu.sync_copy(zer_h.at[pl.ds(r0, _NT)], acc.at[pl.ds(r0, _NT)])
        plsc.subcore_barrier()

        lanes = lax.iota(jnp.int32, 16)
        zvec = jnp.zeros((16,), _F32)
        hmasks = [lanes == hd for hd in range(_H)]
        xor_idx = [lanes ^ kk for kk in (8, 4, 2, 1)]
        rot8_idx = (lanes + 8) & 15
        gdn = lax.GatherDimensionNumbers(
            offset_dims=(), collapsed_slice_dims=(0,), start_index_map=(0,))

        def _shuf(x, idx):
            return lax.gather(x, idx[:, None], gdn, (1,),
                              mode=lax.GatherScatterMode.PROMISE_IN_BOUNDS)

        def _allsum(x):
            for xi in xor_idx:
                x = x + _shuf(x, xi)
            return x

        def base_of(j):
            return (wid + 32 * j) * _C

        def issue_idx(j, B):
            srcv, dstv, dstpv = bufs[B][0], bufs[B][1], bufs[B][2]
            isem = bufs[B][9]
            base = base_of(j)
            pltpu.async_copy(src_h.at[pl.ds(base, _C)], srcv, isem)
            pltpu.async_copy(dst_h.at[pl.ds(base, _C)], dstv, isem)
            pltpu.async_copy(dst_h.at[pl.ds(base, _C)],
                             dstpv.at[pl.ds(0, _C)], isem)

        def wait_idx(j, B):
            srcv, dstv, dstpv = bufs[B][0], bufs[B][1], bufs[B][2]
            isem = bufs[B][9]
            base = base_of(j)
            pltpu.make_async_copy(src_h.at[pl.ds(base, _C)], srcv, isem).wait()
            pltpu.make_async_copy(dst_h.at[pl.ds(base, _C)], dstv, isem).wait()
            pltpu.make_async_copy(dst_h.at[pl.ds(base, _C)],
                                  dstpv.at[pl.ds(0, _C)], isem).wait()

        def issue_gathers(j, B):
            srcv, dstv = bufs[B][0], bufs[B][1]
            kv, qv, ev, mzv = bufs[B][4:8]
            gsem = bufs[B][10]
            base = base_of(j)
            pltpu.async_copy(kh_h.at[srcv], kv, gsem)
            pltpu.async_copy(qh_h.at[dstv], qv, gsem)
            pltpu.async_copy(vh_h.at[srcv], mzv.at[pl.ds(0, _C)], gsem)
            pltpu.async_copy(ee_h.at[pl.ds(base, _C)], ev, gsem)

        def wait_gathers(j, B):
            srcv, dstv = bufs[B][0], bufs[B][1]
            kv, qv, ev, mzv = bufs[B][4:8]
            gsem = bufs[B][10]
            base = base_of(j)
            pltpu.make_async_copy(kh_h.at[srcv], kv, gsem).wait()
            pltpu.make_async_copy(qh_h.at[dstv], qv, gsem).wait()
            pltpu.make_async_copy(vh_h.at[srcv], mzv.at[pl.ds(0, _C)],
                                  gsem).wait()
            pltpu.make_async_copy(ee_h.at[pl.ds(base, _C)], ev, gsem).wait()

        def do_scatters(j, B):
            mzidxv, mzv, attnv = bufs[B][3], bufs[B][7], bufs[B][8]
            ssem = bufs[B][11]
            base = base_of(j)
            pltpu.sync_copy(mzv, acc.at[mzidxv], add=True)
            c = pltpu.async_copy(attnv, attn_o.at[pl.ds(base * 8, _C * 8)],
                                 ssem)
            c.wait()

        def compute(B):
            dstpv, mzidxv = bufs[B][2], bufs[B][3]
            kv, qv, ev, mzv, attnv = bufs[B][4:9]
            for o in (0, 16):
                d = dstpv[pl.ds(o, 16)]
                mzidxv[pl.ds(o, 16)] = d
                mzidxv[pl.ds(_C + o, 16)] = _ZB + lax.shift_right_logical(d, 4)

            def edge_body(ei, carry):
                attn_vec = zvec
                for hd in range(_H):
                    sl = pl.ds(hd * 16, 16)
                    w = kv[ei, sl] * qv[ei, sl] * ev[ei, sl]
                    sv = jnp.exp(jnp.clip(_allsum(w), -5.0, 5.0))
                    mzv[ei, sl] = mzv[ei, sl] * sv
                    attn_vec = jnp.where(hmasks[hd], sv, attn_vec)
                parity = ei & 1
                combined = carry + _shuf(attn_vec, rot8_idx)

                @pl.when(parity == 1)
                def _():
                    attnv[pl.ds((ei >> 1) * 16, 16)] = combined
                pf = 1.0 - lax.convert_element_type(parity, _F32)
                new_carry = attn_vec * lax.broadcast_in_dim(pf, (16,), ())
                d0 = dstpv[pl.ds(ei, 16)][0]
                off = (d0 & 15) * 8
                offc = jnp.minimum(off, 112)
                didx = (lanes - lax.broadcast_in_dim(off - offc, (16,), ())) & 15
                store_vec = _shuf(attn_vec, didx)
                for t in range(8):
                    mzv[_C + ei, pl.ds(t * 16, 16)] = zvec
                mzv[_C + ei, pl.ds(offc, 16)] = store_vec
                return new_carry

            lax.fori_loop(0, _C, edge_body, zvec)

        def pair_body(jp, carry):
            j0 = 2 * jp
            issue_idx(j0 + 1, 1)
            wait_gathers(j0, 0)
            compute(0)
            do_scatters(j0, 0)
            wait_idx(j0 + 1, 1)
            issue_gathers(j0 + 1, 1)

            j1 = j0 + 1

            @pl.when(jp < _NPAIR - 1)
            def _():
                issue_idx(j1 + 1, 0)
            wait_gathers(j1, 1)
            compute(1)
            do_scatters(j1, 1)

            @pl.when(jp < _NPAIR - 1)
            def _():
                wait_idx(j1 + 1, 0)
                issue_gathers(j1 + 1, 0)
            return carry

        issue_idx(0, 0)
        wait_idx(0, 0)
        issue_gathers(0, 0)
        lax.fori_loop(0, _NPAIR, pair_body, 0)

        @pl.when(wid < _TCH - 32 * _JFULL)
        def _():
            jt = _JFULL
            issue_idx(jt, 0)
            wait_idx(jt, 0)
            issue_gathers(jt, 0)
            wait_gathers(jt, 0)
            compute(0)
            do_scatters(jt, 0)

        plsc.subcore_barrier()
        pltpu.sync_copy(acc.at[pl.ds(r0, _NT)], wv_o.at[cid, pl.ds(r0, _NT)])

    return k(qh, kh, vh, ee, src, dst, zer)


def kernel(h, p, e, edge_index, Q_w, Q_b, K_w, K_b, E_w, E_b, V_w, V_b):
    W = jnp.concatenate([Q_w, K_w, V_w], axis=1)
    b = jnp.concatenate([Q_b, K_b, V_b])[None, :]
    qh, kh, vh = _qkv_call(h, p, W[:128], W[128:], b)
    ee = _proj_call(e, E_w * 0.25, E_b[None, :] * 0.25)
    src = edge_index[0]
    dst = edge_index[1]
    zer = jnp.zeros((_NAZ, 128), _F32)
    accd, attn_flat = _sc_edge(qh, kh, vh, ee, src, dst, zer)
    S = jnp.repeat(jnp.eye(_H, dtype=_F32), _D, axis=1)
    h_out = _combine_call(accd, accd.reshape(2, _NAZ * 16, 8), S)
    return (h_out.reshape(_N, _H, _D), attn_flat.reshape(_E, _H, 1))


# speedup vs baseline: 24.8328x; 1.0080x over previous
import functools

import jax
import jax.numpy as jnp
from jax import lax
from jax.experimental import pallas as pl
from jax.experimental.pallas import tpu as pltpu
from jax.experimental.pallas import tpu_sc as plsc

_N = 10000
_E = 320000
_H = 8
_D = 16
_DOUT = _H * _D

_C = 32
_NW = 32
_TCH = _E // _C
_JFULL = _TCH // _NW
_NPAIR = _JFULL // 2
_ZB = 10000
_NAZ = 10752
_NT = _NAZ // 16

_F32 = jnp.float32
_HP = jax.lax.Precision.HIGHEST


def _qkv_call(h, p, W1, W2, b):
    n = h.shape[0]
    bm = 400

    def kern(h_ref, p_ref, w1_ref, w2_ref, b_ref, q_ref, k_ref, v_ref):
        acc = (jnp.dot(h_ref[...], w1_ref[...], preferred_element_type=_F32)
               + jnp.dot(p_ref[...], w2_ref[...], preferred_element_type=_F32)
               + b_ref[...])
        q_ref[...] = acc[:, 0:128]
        k_ref[...] = acc[:, 128:256]
        v_ref[...] = acc[:, 256:384]

    return pl.pallas_call(
        kern,
        grid=(n // bm,),
        in_specs=[pl.BlockSpec((bm, 128), lambda i: (i, 0)),
                  pl.BlockSpec((bm, 128), lambda i: (i, 0)),
                  pl.BlockSpec((128, 384), lambda i: (0, 0)),
                  pl.BlockSpec((128, 384), lambda i: (0, 0)),
                  pl.BlockSpec((1, 384), lambda i: (0, 0))],
        out_specs=[pl.BlockSpec((bm, 128), lambda i: (i, 0))] * 3,
        out_shape=[jax.ShapeDtypeStruct((n, 128), _F32)] * 3,
    )(h, p, W1, W2, b)


def _proj_call(e, W, b):
    m = e.shape[0]
    bm = 1280

    def kern(e_ref, w_ref, b_ref, o_ref):
        o_ref[...] = jnp.dot(e_ref[...], w_ref[...],
                             preferred_element_type=_F32) + b_ref[...]

    return pl.pallas_call(
        kern,
        grid=(m // bm,),
        in_specs=[pl.BlockSpec((bm, 128), lambda i: (i, 0)),
                  pl.BlockSpec((128, 128), lambda i: (0, 0)),
                  pl.BlockSpec((1, 128), lambda i: (0, 0))],
        out_specs=pl.BlockSpec((bm, 128), lambda i: (i, 0)),
        out_shape=jax.ShapeDtypeStruct((m, 128), _F32),
    )(e, W, b)


def _combine_call(wvp, zp, S):
    bm = 400

    def kern(wv_ref, z_ref, s_ref, o_ref):
        wv = wv_ref[0] + wv_ref[1]
        z = z_ref[0] + z_ref[1]
        zr = jnp.dot(z, s_ref[...], preferred_element_type=_F32, precision=_HP)
        o_ref[...] = wv / (zr + 1e-6)

    return pl.pallas_call(
        kern,
        grid=(_N // bm,),
        in_specs=[pl.BlockSpec((2, bm, 128), lambda i: (0, i, 0)),
                  pl.BlockSpec((2, bm, 8), lambda i: (0, i + _ZB * 16 // bm, 0)),
                  pl.BlockSpec((8, 128), lambda i: (0, 0))],
        out_specs=pl.BlockSpec((bm, 128), lambda i: (i, 0)),
        out_shape=jax.ShapeDtypeStruct((_N, 128), _F32),
    )(wvp, zp, S)


def _sc_edge(qh, kh, vh, ee, src, dst, zer):
    mesh = plsc.VectorSubcoreMesh(core_axis_name="c", subcore_axis_name="s")

    def _bufset():
        return [
            pltpu.VMEM((_C,), jnp.int32),
            pltpu.VMEM((_C,), jnp.int32),
            pltpu.VMEM((_C + 16,), jnp.int32),
            pltpu.VMEM((2 * _C,), jnp.int32),
            pltpu.VMEM((_C, 128), _F32),
            pltpu.VMEM((_C, 128), _F32),
            pltpu.VMEM((_C, 128), _F32),
            pltpu.VMEM((2 * _C, 128), _F32),
            pltpu.VMEM((_C * 8,), _F32),
            pltpu.SemaphoreType.DMA,
            pltpu.SemaphoreType.DMA,
            pltpu.SemaphoreType.DMA,
        ]

    @functools.partial(
        pl.kernel, mesh=mesh,
        out_type=[jax.ShapeDtypeStruct((2, _NAZ, 128), _F32),
                  jax.ShapeDtypeStruct((_E * 8,), _F32)],
        scratch_types=(_bufset() + _bufset()
                       + [pltpu.VMEM_SHARED((_NAZ, 128), _F32)]),
    )
    def k(qh_h, kh_h, vh_h, ee_h, src_h, dst_h, zer_h, wv_o, attn_o, *sc):
        bufs = (sc[0:12], sc[12:24])
        acc = sc[24]
        cid = lax.axis_index("c")
        sid = lax.axis_index("s")
        wid = sid * 2 + cid
        r0 = sid * _NT

        pltpu.sync_copy(zer_h.at[pl.ds(r0, _NT)], acc.at[pl.ds(r0, _NT)])
        plsc.subcore_barrier()

        lanes = lax.iota(jnp.int32, 16)
        zvec = jnp.zeros((16,), _F32)
        hmasks = [lanes == hd for hd in range(_H)]
        xor_idx = [lanes ^ kk for kk in (8, 4, 2, 1)]
        rot8_idx = (lanes + 8) & 15
        gdn = lax.GatherDimensionNumbers(
            offset_dims=(), collapsed_slice_dims=(0,), start_index_map=(0,))

        def _shuf(x, idx):
            return lax.gather(x, idx[:, None], gdn, (1,),
                              mode=lax.GatherScatterMode.PROMISE_IN_BOUNDS)

        def _allsum(x):
            for xi in xor_idx:
                x = x + _shuf(x, xi)
            return x

        def base_of(j):
            return (wid + 32 * j) * _C

        def issue_idx(j, B):
            srcv, dstv, dstpv = bufs[B][0], bufs[B][1], bufs[B][2]
            isem = bufs[B][9]
            base = base_of(j)
            pltpu.async_copy(src_h.at[pl.ds(base, _C)], srcv, isem)
            pltpu.async_copy(dst_h.at[pl.ds(base, _C)], dstv, isem)
            pltpu.async_copy(dst_h.at[pl.ds(base, _C)],
                             dstpv.at[pl.ds(0, _C)], isem)

        def wait_idx(j, B):
            srcv, dstv, dstpv = bufs[B][0], bufs[B][1], bufs[B][2]
            isem = bufs[B][9]
            base = base_of(j)
            pltpu.make_async_copy(src_h.at[pl.ds(base, _C)], srcv, isem).wait()
            pltpu.make_async_copy(dst_h.at[pl.ds(base, _C)], dstv, isem).wait()
            pltpu.make_async_copy(dst_h.at[pl.ds(base, _C)],
                                  dstpv.at[pl.ds(0, _C)], isem).wait()

        def issue_gathers(j, B):
            srcv, dstv = bufs[B][0], bufs[B][1]
            kv, qv, ev, mzv = bufs[B][4:8]
            gsem = bufs[B][10]
            base = base_of(j)
            pltpu.async_copy(kh_h.at[srcv], kv, gsem)
            pltpu.async_copy(qh_h.at[dstv], qv, gsem)
            pltpu.async_copy(vh_h.at[srcv], mzv.at[pl.ds(0, _C)], gsem)
            pltpu.async_copy(ee_h.at[pl.ds(base, _C)], ev, gsem)

        def wait_gathers(j, B):
            srcv, dstv = bufs[B][0], bufs[B][1]
            kv, qv, ev, mzv = bufs[B][4:8]
            gsem = bufs[B][10]
            base = base_of(j)
            pltpu.make_async_copy(kh_h.at[srcv], kv, gsem).wait()
            pltpu.make_async_copy(qh_h.at[dstv], qv, gsem).wait()
            pltpu.make_async_copy(vh_h.at[srcv], mzv.at[pl.ds(0, _C)],
                                  gsem).wait()
            pltpu.make_async_copy(ee_h.at[pl.ds(base, _C)], ev, gsem).wait()

        def do_scatters(j, B):
            mzidxv, mzv, attnv = bufs[B][3], bufs[B][7], bufs[B][8]
            ssem = bufs[B][11]
            base = base_of(j)
            pltpu.sync_copy(mzv, acc.at[mzidxv], add=True)
            c = pltpu.async_copy(attnv, attn_o.at[pl.ds(base * 8, _C * 8)],
                                 ssem)
            c.wait()

        def compute(B):
            dstpv, mzidxv = bufs[B][2], bufs[B][3]
            kv, qv, ev, mzv, attnv = bufs[B][4:9]
            for o in (0, 16):
                d = dstpv[pl.ds(o, 16)]
                mzidxv[pl.ds(o, 16)] = d
                mzidxv[pl.ds(_C + o, 16)] = _ZB + lax.shift_right_logical(d, 4)

            def edge_body(ei, carry):
                attn_vec = zvec
                for hd in range(_H):
                    sl = pl.ds(hd * 16, 16)
                    w = kv[ei, sl] * qv[ei, sl] * ev[ei, sl]
                    sv = jnp.exp(jnp.clip(_allsum(w), -5.0, 5.0))
                    mzv[ei, sl] = mzv[ei, sl] * sv
                    attn_vec = jnp.where(hmasks[hd], sv, attn_vec)
                parity = ei & 1
                combined = carry + _shuf(attn_vec, rot8_idx)

                @pl.when(parity == 1)
                def _():
                    attnv[pl.ds((ei >> 1) * 16, 16)] = combined
                pf = 1.0 - lax.convert_element_type(parity, _F32)
                new_carry = attn_vec * lax.broadcast_in_dim(pf, (16,), ())
                d0 = dstpv[pl.ds(ei, 16)][0]
                off = (d0 & 15) * 8
                offc = jnp.minimum(off, 112)
                didx = (lanes - lax.broadcast_in_dim(off - offc, (16,), ())) & 15
                store_vec = _shuf(attn_vec, didx)
                for t in range(8):
                    mzv[_C + ei, pl.ds(t * 16, 16)] = zvec
                mzv[_C + ei, pl.ds(offc, 16)] = store_vec
                return new_carry

            def edge_pair(e2, carry):
                carry = edge_body(e2 * 2, carry)
                return edge_body(e2 * 2 + 1, carry)

            lax.fori_loop(0, _C // 2, edge_pair, zvec)

        def pair_body(jp, carry):
            j0 = 2 * jp
            issue_idx(j0 + 1, 1)
            wait_gathers(j0, 0)
            compute(0)
            do_scatters(j0, 0)
            wait_idx(j0 + 1, 1)
            issue_gathers(j0 + 1, 1)

            j1 = j0 + 1

            @pl.when(jp < _NPAIR - 1)
            def _():
                issue_idx(j1 + 1, 0)
            wait_gathers(j1, 1)
            compute(1)
            do_scatters(j1, 1)

            @pl.when(jp < _NPAIR - 1)
            def _():
                wait_idx(j1 + 1, 0)
                issue_gathers(j1 + 1, 0)
            return carry

        issue_idx(0, 0)
        wait_idx(0, 0)
        issue_gathers(0, 0)
        lax.fori_loop(0, _NPAIR, pair_body, 0)

        @pl.when(wid < _TCH - 32 * _JFULL)
        def _():
            jt = _JFULL
            issue_idx(jt, 0)
            wait_idx(jt, 0)
            issue_gathers(jt, 0)
            wait_gathers(jt, 0)
            compute(0)
            do_scatters(jt, 0)

        plsc.subcore_barrier()
        pltpu.sync_copy(acc.at[pl.ds(r0, _NT)], wv_o.at[cid, pl.ds(r0, _NT)])

    return k(qh, kh, vh, ee, src, dst, zer)


def kernel(h, p, e, edge_index, Q_w, Q_b, K_w, K_b, E_w, E_b, V_w, V_b):
    W = jnp.concatenate([Q_w, K_w, V_w], axis=1)
    b = jnp.concatenate([Q_b, K_b, V_b])[None, :]
    qh, kh, vh = _qkv_call(h, p, W[:128], W[128:], b)
    ee = _proj_call(e, E_w * 0.25, E_b[None, :] * 0.25)
    src = edge_index[0]
    dst = edge_index[1]
    zer = jnp.zeros((_NAZ, 128), _F32)
    accd, attn_flat = _sc_edge(qh, kh, vh, ee, src, dst, zer)
    S = jnp.repeat(jnp.eye(_H, dtype=_F32), _D, axis=1)
    h_out = _combine_call(accd, accd.reshape(2, _NAZ * 16, 8), S)
    return (h_out.reshape(_N, _H, _D), attn_flat.reshape(_E, _H, 1))


# speedup vs baseline: 24.8965x; 1.0026x over previous
import functools

import jax
import jax.numpy as jnp
from jax import lax
from jax.experimental import pallas as pl
from jax.experimental.pallas import tpu as pltpu
from jax.experimental.pallas import tpu_sc as plsc

_N = 10000
_E = 320000
_H = 8
_D = 16
_DOUT = _H * _D

_C = 32
_NW = 32
_TCH = _E // _C
_JFULL = _TCH // _NW
_NPAIR = _JFULL // 2
_ZB = 10000
_NAZ = 10752
_NT = _NAZ // 16

_F32 = jnp.float32
_HP = jax.lax.Precision.HIGHEST


def _qkv_call(h, p, W1, W2, b):
    n = h.shape[0]
    bm = 400

    def kern(h_ref, p_ref, w1_ref, w2_ref, b_ref, q_ref, k_ref, v_ref):
        acc = (jnp.dot(h_ref[...], w1_ref[...], preferred_element_type=_F32)
               + jnp.dot(p_ref[...], w2_ref[...], preferred_element_type=_F32)
               + b_ref[...])
        q_ref[...] = acc[:, 0:128]
        k_ref[...] = acc[:, 128:256]
        v_ref[...] = acc[:, 256:384]

    return pl.pallas_call(
        kern,
        grid=(n // bm,),
        in_specs=[pl.BlockSpec((bm, 128), lambda i: (i, 0)),
                  pl.BlockSpec((bm, 128), lambda i: (i, 0)),
                  pl.BlockSpec((128, 384), lambda i: (0, 0)),
                  pl.BlockSpec((128, 384), lambda i: (0, 0)),
                  pl.BlockSpec((1, 384), lambda i: (0, 0))],
        out_specs=[pl.BlockSpec((bm, 128), lambda i: (i, 0))] * 3,
        out_shape=[jax.ShapeDtypeStruct((n, 128), _F32)] * 3,
    )(h, p, W1, W2, b)


def _proj_call(e, W, b):
    m = e.shape[0]
    bm = 1280

    def kern(e_ref, w_ref, b_ref, o_ref):
        o_ref[...] = jnp.dot(e_ref[...], w_ref[...],
                             preferred_element_type=_F32) + b_ref[...]

    return pl.pallas_call(
        kern,
        grid=(m // bm,),
        in_specs=[pl.BlockSpec((bm, 128), lambda i: (i, 0)),
                  pl.BlockSpec((128, 128), lambda i: (0, 0)),
                  pl.BlockSpec((1, 128), lambda i: (0, 0))],
        out_specs=pl.BlockSpec((bm, 128), lambda i: (i, 0)),
        out_shape=jax.ShapeDtypeStruct((m, 128), _F32),
    )(e, W, b)


def _combine_call(wvp, zp, S):
    bm = 400

    def kern(wv_ref, z_ref, s_ref, o_ref):
        wv = wv_ref[0] + wv_ref[1]
        z = z_ref[0] + z_ref[1]
        zr = jnp.dot(z, s_ref[...], preferred_element_type=_F32, precision=_HP)
        o_ref[...] = wv / (zr + 1e-6)

    return pl.pallas_call(
        kern,
        grid=(_N // bm,),
        in_specs=[pl.BlockSpec((2, bm, 128), lambda i: (0, i, 0)),
                  pl.BlockSpec((2, bm, 8), lambda i: (0, i + _ZB * 16 // bm, 0)),
                  pl.BlockSpec((8, 128), lambda i: (0, 0))],
        out_specs=pl.BlockSpec((bm, 128), lambda i: (i, 0)),
        out_shape=jax.ShapeDtypeStruct((_N, 128), _F32),
    )(wvp, zp, S)


def _sc_edge(qh, kh, vh, ee, src, dst, zer):
    mesh = plsc.VectorSubcoreMesh(core_axis_name="c", subcore_axis_name="s")

    def _bufset():
        return [
            pltpu.VMEM((_C,), jnp.int32),
            pltpu.VMEM((_C,), jnp.int32),
            pltpu.VMEM((_C + 16,), jnp.int32),
            pltpu.VMEM((2 * _C,), jnp.int32),
            pltpu.VMEM((_C, 128), _F32),
            pltpu.VMEM((_C, 128), _F32),
            pltpu.VMEM((_C, 128), _F32),
            pltpu.VMEM((2 * _C, 128), _F32),
            pltpu.VMEM((_C * 8,), _F32),
            pltpu.SemaphoreType.DMA,
            pltpu.SemaphoreType.DMA,
            pltpu.SemaphoreType.DMA,
        ]

    @functools.partial(
        pl.kernel, mesh=mesh,
        out_type=[jax.ShapeDtypeStruct((2, _NAZ, 128), _F32),
                  jax.ShapeDtypeStruct((_E * 8,), _F32)],
        scratch_types=(_bufset() + _bufset()
                       + [pltpu.VMEM_SHARED((_NAZ, 128), _F32)]),
    )
    def k(qh_h, kh_h, vh_h, ee_h, src_h, dst_h, zer_h, wv_o, attn_o, *sc):
        bufs = (sc[0:12], sc[12:24])
        acc = sc[24]
        cid = lax.axis_index("c")
        sid = lax.axis_index("s")
        wid = sid * 2 + cid
        r0 = sid * _NT

        pltpu.sync_copy(zer_h.at[pl.ds(r0, _NT)], acc.at[pl.ds(r0, _NT)])
        plsc.subcore_barrier()

        lanes = lax.iota(jnp.int32, 16)
        zvec = jnp.zeros((16,), _F32)
        onev = jnp.ones((16,), _F32)
        hmasks_f = [jnp.where(lanes == hd, onev, zvec) for hd in range(_H)]
        xor_idx = [lanes ^ kk for kk in (8, 4, 2, 1)]
        rot8_idx = (lanes + 8) & 15
        gdn = lax.GatherDimensionNumbers(
            offset_dims=(), collapsed_slice_dims=(0,), start_index_map=(0,))

        def _shuf(x, idx):
            return lax.gather(x, idx[:, None], gdn, (1,),
                              mode=lax.GatherScatterMode.PROMISE_IN_BOUNDS)

        def _allsum(x):
            for xi in xor_idx:
                x = x + _shuf(x, xi)
            return x

        def base_of(j):
            return (wid + 32 * j) * _C

        def issue_idx(j, B):
            srcv, dstv, dstpv = bufs[B][0], bufs[B][1], bufs[B][2]
            isem = bufs[B][9]
            base = base_of(j)
            pltpu.async_copy(src_h.at[pl.ds(base, _C)], srcv, isem)
            pltpu.async_copy(dst_h.at[pl.ds(base, _C)], dstv, isem)
            pltpu.async_copy(dst_h.at[pl.ds(base, _C)],
                             dstpv.at[pl.ds(0, _C)], isem)

        def wait_idx(j, B):
            srcv, dstv, dstpv = bufs[B][0], bufs[B][1], bufs[B][2]
            isem = bufs[B][9]
            base = base_of(j)
            pltpu.make_async_copy(src_h.at[pl.ds(base, _C)], srcv, isem).wait()
            pltpu.make_async_copy(dst_h.at[pl.ds(base, _C)], dstv, isem).wait()
            pltpu.make_async_copy(dst_h.at[pl.ds(base, _C)],
                                  dstpv.at[pl.ds(0, _C)], isem).wait()

        def issue_gathers(j, B):
            srcv, dstv = bufs[B][0], bufs[B][1]
            kv, qv, ev, mzv = bufs[B][4:8]
            gsem = bufs[B][10]
            base = base_of(j)
            pltpu.async_copy(kh_h.at[srcv], kv, gsem)
            pltpu.async_copy(qh_h.at[dstv], qv, gsem)
            pltpu.async_copy(vh_h.at[srcv], mzv.at[pl.ds(0, _C)], gsem)
            pltpu.async_copy(ee_h.at[pl.ds(base, _C)], ev, gsem)

        def wait_gathers(j, B):
            srcv, dstv = bufs[B][0], bufs[B][1]
            kv, qv, ev, mzv = bufs[B][4:8]
            gsem = bufs[B][10]
            base = base_of(j)
            pltpu.make_async_copy(kh_h.at[srcv], kv, gsem).wait()
            pltpu.make_async_copy(qh_h.at[dstv], qv, gsem).wait()
            pltpu.make_async_copy(vh_h.at[srcv], mzv.at[pl.ds(0, _C)],
                                  gsem).wait()
            pltpu.make_async_copy(ee_h.at[pl.ds(base, _C)], ev, gsem).wait()

        def do_scatters(j, B):
            mzidxv, mzv, attnv = bufs[B][3], bufs[B][7], bufs[B][8]
            ssem = bufs[B][11]
            base = base_of(j)
            pltpu.sync_copy(mzv, acc.at[mzidxv], add=True)
            c = pltpu.async_copy(attnv, attn_o.at[pl.ds(base * 8, _C * 8)],
                                 ssem)
            c.wait()

        def compute(B):
            dstpv, mzidxv = bufs[B][2], bufs[B][3]
            kv, qv, ev, mzv, attnv = bufs[B][4:9]
            for o in (0, 16):
                d = dstpv[pl.ds(o, 16)]
                mzidxv[pl.ds(o, 16)] = d
                mzidxv[pl.ds(_C + o, 16)] = _ZB + lax.shift_right_logical(d, 4)

            def do_edge(ei):
                attn_vec = zvec
                for hd in range(_H):
                    sl = pl.ds(hd * 16, 16)
                    w = kv[ei, sl] * qv[ei, sl] * ev[ei, sl]
                    sv = jnp.exp(jnp.clip(_allsum(w), -5.0, 5.0))
                    mzv[ei, sl] = mzv[ei, sl] * sv
                    attn_vec = attn_vec + sv * hmasks_f[hd]
                d0 = dstpv[pl.ds(ei, 16)][0]
                off = (d0 & 15) * 8
                offc = jnp.minimum(off, 112)
                didx = (lanes - lax.broadcast_in_dim(off - offc, (16,), ())) & 15
                store_vec = _shuf(attn_vec, didx)
                for t in range(8):
                    mzv[_C + ei, pl.ds(t * 16, 16)] = zvec
                mzv[_C + ei, pl.ds(offc, 16)] = store_vec
                return attn_vec

            def edge_pair(e2, carry):
                eA = e2 * 2
                aA = do_edge(eA)
                aB = do_edge(eA + 1)
                attnv[pl.ds(e2 * 16, 16)] = aA + _shuf(aB, rot8_idx)
                return carry

            lax.fori_loop(0, _C // 2, edge_pair, 0)

        def pair_body(jp, carry):
            j0 = 2 * jp
            issue_idx(j0 + 1, 1)
            wait_gathers(j0, 0)
            compute(0)
            do_scatters(j0, 0)
            wait_idx(j0 + 1, 1)
            issue_gathers(j0 + 1, 1)

            j1 = j0 + 1

            @pl.when(jp < _NPAIR - 1)
            def _():
                issue_idx(j1 + 1, 0)
            wait_gathers(j1, 1)
            compute(1)
            do_scatters(j1, 1)

            @pl.when(jp < _NPAIR - 1)
            def _():
                wait_idx(j1 + 1, 0)
                issue_gathers(j1 + 1, 0)
            return carry

        issue_idx(0, 0)
        wait_idx(0, 0)
        issue_gathers(0, 0)
        lax.fori_loop(0, _NPAIR, pair_body, 0)

        @pl.when(wid < _TCH - 32 * _JFULL)
        def _():
            jt = _JFULL
            issue_idx(jt, 0)
            wait_idx(jt, 0)
            issue_gathers(jt, 0)
            wait_gathers(jt, 0)
            compute(0)
            do_scatters(jt, 0)

        plsc.subcore_barrier()
        pltpu.sync_copy(acc.at[pl.ds(r0, _NT)], wv_o.at[cid, pl.ds(r0, _NT)])

    return k(qh, kh, vh, ee, src, dst, zer)


def kernel(h, p, e, edge_index, Q_w, Q_b, K_w, K_b, E_w, E_b, V_w, V_b):
    W = jnp.concatenate([Q_w, K_w, V_w], axis=1)
    b = jnp.concatenate([Q_b, K_b, V_b])[None, :]
    qh, kh, vh = _qkv_call(h, p, W[:128], W[128:], b)
    ee = _proj_call(e, E_w * 0.25, E_b[None, :] * 0.25)
    src = edge_index[0]
    dst = edge_index[1]
    zer = jnp.zeros((_NAZ, 128), _F32)
    accd, attn_flat = _sc_edge(qh, kh, vh, ee, src, dst, zer)
    S = jnp.repeat(jnp.eye(_H, dtype=_F32), _D, axis=1)
    h_out = _combine_call(accd, accd.reshape(2, _NAZ * 16, 8), S)
    return (h_out.reshape(_N, _H, _D), attn_flat.reshape(_E, _H, 1))


# speedup vs baseline: 26.1954x; 1.0522x over previous
import functools

import jax
import jax.numpy as jnp
from jax import lax
from jax.experimental import pallas as pl
from jax.experimental.pallas import tpu as pltpu
from jax.experimental.pallas import tpu_sc as plsc

_N = 10000
_E = 320000
_H = 8
_D = 16
_DOUT = _H * _D

_C = 32
_NW = 32
_TCH = _E // _C
_JFULL = _TCH // _NW
_NPAIR = _JFULL // 2
_ZB = 10000
_NAZ = 10752
_NT = _NAZ // 16

_F32 = jnp.float32
_HP = jax.lax.Precision.HIGHEST


def _qkv_call(h, p, W1, W2, b):
    n = h.shape[0]
    bm = 400

    def kern(h_ref, p_ref, w1_ref, w2_ref, b_ref, q_ref, k_ref, v_ref):
        acc = (jnp.dot(h_ref[...], w1_ref[...], preferred_element_type=_F32)
               + jnp.dot(p_ref[...], w2_ref[...], preferred_element_type=_F32)
               + b_ref[...])
        q_ref[...] = acc[:, 0:128]
        k_ref[...] = acc[:, 128:256]
        v_ref[...] = acc[:, 256:384]

    return pl.pallas_call(
        kern,
        grid=(n // bm,),
        in_specs=[pl.BlockSpec((bm, 128), lambda i: (i, 0)),
                  pl.BlockSpec((bm, 128), lambda i: (i, 0)),
                  pl.BlockSpec((128, 384), lambda i: (0, 0)),
                  pl.BlockSpec((128, 384), lambda i: (0, 0)),
                  pl.BlockSpec((1, 384), lambda i: (0, 0))],
        out_specs=[pl.BlockSpec((bm, 128), lambda i: (i, 0))] * 3,
        out_shape=[jax.ShapeDtypeStruct((n, 128), _F32)] * 3,
    )(h, p, W1, W2, b)


def _proj_call(e, W, b):
    m = e.shape[0]
    bm = 1280

    def kern(e_ref, w_ref, b_ref, o_ref):
        o_ref[...] = jnp.dot(e_ref[...], w_ref[...],
                             preferred_element_type=_F32) + b_ref[...]

    return pl.pallas_call(
        kern,
        grid=(m // bm,),
        in_specs=[pl.BlockSpec((bm, 128), lambda i: (i, 0)),
                  pl.BlockSpec((128, 128), lambda i: (0, 0)),
                  pl.BlockSpec((1, 128), lambda i: (0, 0))],
        out_specs=pl.BlockSpec((bm, 128), lambda i: (i, 0)),
        out_shape=jax.ShapeDtypeStruct((m, 128), _F32),
    )(e, W, b)


def _combine_call(wvp, zp, S):
    bm = 400

    def kern(wv_ref, z_ref, s_ref, o_ref):
        wv = wv_ref[0] + wv_ref[1]
        z = z_ref[0] + z_ref[1]
        zr = jnp.dot(z, s_ref[...], preferred_element_type=_F32, precision=_HP)
        o_ref[...] = wv / (zr + 1e-6)

    return pl.pallas_call(
        kern,
        grid=(_N // bm,),
        in_specs=[pl.BlockSpec((2, bm, 128), lambda i: (0, i, 0)),
                  pl.BlockSpec((2, bm, 8), lambda i: (0, i + _ZB * 16 // bm, 0)),
                  pl.BlockSpec((8, 128), lambda i: (0, 0))],
        out_specs=pl.BlockSpec((bm, 128), lambda i: (i, 0)),
        out_shape=jax.ShapeDtypeStruct((_N, 128), _F32),
    )(wvp, zp, S)


def _sc_edge(qh, kh, vh, ee, src, dst, zer):
    mesh = plsc.VectorSubcoreMesh(core_axis_name="c", subcore_axis_name="s")

    def _bufset():
        return [
            pltpu.VMEM((_C,), jnp.int32),
            pltpu.VMEM((_C,), jnp.int32),
            pltpu.VMEM((_C + 16,), jnp.int32),
            pltpu.VMEM((2 * _C,), jnp.int32),
            pltpu.VMEM((_C, 128), _F32),
            pltpu.VMEM((_C, 128), _F32),
            pltpu.VMEM((_C, 128), _F32),
            pltpu.VMEM((2 * _C, 128), _F32),
            pltpu.VMEM((_C * 8,), _F32),
            pltpu.SemaphoreType.DMA,
            pltpu.SemaphoreType.DMA,
            pltpu.SemaphoreType.DMA,
        ]

    @functools.partial(
        pl.kernel, mesh=mesh,
        out_type=[jax.ShapeDtypeStruct((2, _NAZ, 128), _F32),
                  jax.ShapeDtypeStruct((_E * 8,), _F32)],
        scratch_types=(_bufset() + _bufset()
                       + [pltpu.VMEM_SHARED((_NAZ, 128), _F32)]),
    )
    def k(qh_h, kh_h, vh_h, ee_h, src_h, dst_h, zer_h, wv_o, attn_o, *sc):
        bufs = (sc[0:12], sc[12:24])
        acc = sc[24]
        cid = lax.axis_index("c")
        sid = lax.axis_index("s")
        wid = sid * 2 + cid
        r0 = sid * _NT

        pltpu.sync_copy(zer_h.at[pl.ds(r0, _NT)], acc.at[pl.ds(r0, _NT)])
        plsc.subcore_barrier()

        lanes = lax.iota(jnp.int32, 16)
        zvec = jnp.zeros((16,), _F32)
        onev = jnp.ones((16,), _F32)
        hmasks_f = [jnp.where(lanes == hd, onev, zvec) for hd in range(_H)]
        xor_idx = [lanes ^ kk for kk in (8, 4, 2, 1)]
        rot8_idx = (lanes + 8) & 15
        gdn = lax.GatherDimensionNumbers(
            offset_dims=(), collapsed_slice_dims=(0,), start_index_map=(0,))

        def _shuf(x, idx):
            return lax.gather(x, idx[:, None], gdn, (1,),
                              mode=lax.GatherScatterMode.PROMISE_IN_BOUNDS)

        def _allsum(x):
            for xi in xor_idx:
                x = x + _shuf(x, xi)
            return x

        def base_of(j):
            return (wid + 32 * j) * _C

        def issue_idx(j, B):
            srcv, dstv, dstpv = bufs[B][0], bufs[B][1], bufs[B][2]
            isem = bufs[B][9]
            base = base_of(j)
            pltpu.async_copy(src_h.at[pl.ds(base, _C)], srcv, isem)
            pltpu.async_copy(dst_h.at[pl.ds(base, _C)], dstv, isem)
            pltpu.async_copy(dst_h.at[pl.ds(base, _C)],
                             dstpv.at[pl.ds(0, _C)], isem)

        def wait_idx(j, B):
            srcv, dstv, dstpv = bufs[B][0], bufs[B][1], bufs[B][2]
            isem = bufs[B][9]
            base = base_of(j)
            pltpu.make_async_copy(src_h.at[pl.ds(base, _C)], srcv, isem).wait()
            pltpu.make_async_copy(dst_h.at[pl.ds(base, _C)], dstv, isem).wait()
            pltpu.make_async_copy(dst_h.at[pl.ds(base, _C)],
                                  dstpv.at[pl.ds(0, _C)], isem).wait()

        def issue_gathers(j, B):
            srcv, dstv = bufs[B][0], bufs[B][1]
            kv, qv, ev, mzv = bufs[B][4:8]
            gsem = bufs[B][10]
            base = base_of(j)
            pltpu.async_copy(kh_h.at[srcv], kv, gsem)
            pltpu.async_copy(qh_h.at[dstv], qv, gsem)
            pltpu.async_copy(vh_h.at[srcv], mzv.at[pl.ds(0, _C)], gsem)
            pltpu.async_copy(ee_h.at[pl.ds(base, _C)], ev, gsem)

        def wait_gathers(j, B):
            srcv, dstv = bufs[B][0], bufs[B][1]
            kv, qv, ev, mzv = bufs[B][4:8]
            gsem = bufs[B][10]
            base = base_of(j)
            pltpu.make_async_copy(kh_h.at[srcv], kv, gsem).wait()
            pltpu.make_async_copy(qh_h.at[dstv], qv, gsem).wait()
            pltpu.make_async_copy(vh_h.at[srcv], mzv.at[pl.ds(0, _C)],
                                  gsem).wait()
            pltpu.make_async_copy(ee_h.at[pl.ds(base, _C)], ev, gsem).wait()

        def do_scatters(j, B):
            mzidxv, mzv, attnv = bufs[B][3], bufs[B][7], bufs[B][8]
            ssem = bufs[B][11]
            base = base_of(j)
            pltpu.sync_copy(mzv, acc.at[mzidxv], add=True)
            c = pltpu.async_copy(attnv, attn_o.at[pl.ds(base * 8, _C * 8)],
                                 ssem)
            c.wait()

        def compute(B):
            dstpv, mzidxv = bufs[B][2], bufs[B][3]
            kv, qv, ev, mzv, attnv = bufs[B][4:9]
            for o in (0, 16):
                d = dstpv[pl.ds(o, 16)]
                mzidxv[pl.ds(o, 16)] = d
                mzidxv[pl.ds(_C + o, 16)] = _ZB + lax.shift_right_logical(d, 4)

            def do_edge(ei):
                attn_vec = zvec
                for hd in range(_H):
                    sl = pl.ds(hd * 16, 16)
                    w = kv[ei, sl] * qv[ei, sl] * ev[ei, sl]
                    sv = jnp.exp(jnp.clip(_allsum(w), -5.0, 5.0))
                    mzv[ei, sl] = mzv[ei, sl] * sv
                    attn_vec = attn_vec + sv * hmasks_f[hd]
                d0 = dstpv[pl.ds(ei, 16)][0]
                off = (d0 & 15) * 8
                offc = jnp.minimum(off, 112)
                didx = (lanes - lax.broadcast_in_dim(off - offc, (16,), ())) & 15
                store_vec = _shuf(attn_vec, didx)
                for t in range(8):
                    mzv[_C + ei, pl.ds(t * 16, 16)] = zvec
                mzv[_C + ei, pl.ds(offc, 16)] = store_vec
                return attn_vec

            @plsc.parallel_loop(0, _C // 2, unroll=2)
            def _(e2):
                eA = e2 * 2
                aA = do_edge(eA)
                aB = do_edge(eA + 1)
                attnv[pl.ds(e2 * 16, 16)] = aA + _shuf(aB, rot8_idx)

        def pair_body(jp, carry):
            j0 = 2 * jp
            issue_idx(j0 + 1, 1)
            wait_gathers(j0, 0)
            compute(0)
            do_scatters(j0, 0)
            wait_idx(j0 + 1, 1)
            issue_gathers(j0 + 1, 1)

            j1 = j0 + 1

            @pl.when(jp < _NPAIR - 1)
            def _():
                issue_idx(j1 + 1, 0)
            wait_gathers(j1, 1)
            compute(1)
            do_scatters(j1, 1)

            @pl.when(jp < _NPAIR - 1)
            def _():
                wait_idx(j1 + 1, 0)
                issue_gathers(j1 + 1, 0)
            return carry

        issue_idx(0, 0)
        wait_idx(0, 0)
        issue_gathers(0, 0)
        lax.fori_loop(0, _NPAIR, pair_body, 0)

        @pl.when(wid < _TCH - 32 * _JFULL)
        def _():
            jt = _JFULL
            issue_idx(jt, 0)
            wait_idx(jt, 0)
            issue_gathers(jt, 0)
            wait_gathers(jt, 0)
            compute(0)
            do_scatters(jt, 0)

        plsc.subcore_barrier()
        pltpu.sync_copy(acc.at[pl.ds(r0, _NT)], wv_o.at[cid, pl.ds(r0, _NT)])

    return k(qh, kh, vh, ee, src, dst, zer)


def kernel(h, p, e, edge_index, Q_w, Q_b, K_w, K_b, E_w, E_b, V_w, V_b):
    W = jnp.concatenate([Q_w, K_w, V_w], axis=1)
    b = jnp.concatenate([Q_b, K_b, V_b])[None, :]
    qh, kh, vh = _qkv_call(h, p, W[:128], W[128:], b)
    ee = _proj_call(e, E_w * 0.25, E_b[None, :] * 0.25)
    src = edge_index[0]
    dst = edge_index[1]
    zer = jnp.zeros((_NAZ, 128), _F32)
    accd, attn_flat = _sc_edge(qh, kh, vh, ee, src, dst, zer)
    S = jnp.repeat(jnp.eye(_H, dtype=_F32), _D, axis=1)
    h_out = _combine_call(accd, accd.reshape(2, _NAZ * 16, 8), S)
    return (h_out.reshape(_N, _H, _D), attn_flat.reshape(_E, _H, 1))


# speedup vs baseline: 26.4257x; 1.0088x over previous
import functools

import jax
import jax.numpy as jnp
from jax import lax
from jax.experimental import pallas as pl
from jax.experimental.pallas import tpu as pltpu
from jax.experimental.pallas import tpu_sc as plsc

_N = 10000
_E = 320000
_H = 8
_D = 16
_DOUT = _H * _D

_C = 32
_NW = 32
_TCH = _E // _C
_JFULL = _TCH // _NW
_NPAIR = _JFULL // 2
_ZB = 10000
_NAZ = 10752
_NT = _NAZ // 16

_F32 = jnp.float32
_HP = jax.lax.Precision.HIGHEST


def _qkv_call(h, p, W1, W2, b):
    n = h.shape[0]
    bm = 400

    def kern(h_ref, p_ref, w1_ref, w2_ref, b_ref, q_ref, k_ref, v_ref):
        acc = (jnp.dot(h_ref[...], w1_ref[...], preferred_element_type=_F32)
               + jnp.dot(p_ref[...], w2_ref[...], preferred_element_type=_F32)
               + b_ref[...])
        q_ref[...] = acc[:, 0:128]
        k_ref[...] = acc[:, 128:256]
        v_ref[...] = acc[:, 256:384]

    return pl.pallas_call(
        kern,
        grid=(n // bm,),
        in_specs=[pl.BlockSpec((bm, 128), lambda i: (i, 0)),
                  pl.BlockSpec((bm, 128), lambda i: (i, 0)),
                  pl.BlockSpec((128, 384), lambda i: (0, 0)),
                  pl.BlockSpec((128, 384), lambda i: (0, 0)),
                  pl.BlockSpec((1, 384), lambda i: (0, 0))],
        out_specs=[pl.BlockSpec((bm, 128), lambda i: (i, 0))] * 3,
        out_shape=[jax.ShapeDtypeStruct((n, 128), _F32)] * 3,
    )(h, p, W1, W2, b)


def _proj_call(e, W, b):
    m = e.shape[0]
    bm = 1280

    def kern(e_ref, w_ref, b_ref, o_ref):
        o_ref[...] = jnp.dot(e_ref[...], w_ref[...],
                             preferred_element_type=_F32) + b_ref[...]

    return pl.pallas_call(
        kern,
        grid=(m // bm,),
        in_specs=[pl.BlockSpec((bm, 128), lambda i: (i, 0)),
                  pl.BlockSpec((128, 128), lambda i: (0, 0)),
                  pl.BlockSpec((1, 128), lambda i: (0, 0))],
        out_specs=pl.BlockSpec((bm, 128), lambda i: (i, 0)),
        out_shape=jax.ShapeDtypeStruct((m, 128), _F32),
    )(e, W, b)


def _combine_call(wvp, zp, S):
    bm = 400

    def kern(wv_ref, z_ref, s_ref, o_ref):
        wv = wv_ref[0] + wv_ref[1]
        z = z_ref[0] + z_ref[1]
        zr = jnp.dot(z, s_ref[...], preferred_element_type=_F32, precision=_HP)
        o_ref[...] = wv / (zr + 1e-6)

    return pl.pallas_call(
        kern,
        grid=(_N // bm,),
        in_specs=[pl.BlockSpec((2, bm, 128), lambda i: (0, i, 0)),
                  pl.BlockSpec((2, bm, 8), lambda i: (0, i + _ZB * 16 // bm, 0)),
                  pl.BlockSpec((8, 128), lambda i: (0, 0))],
        out_specs=pl.BlockSpec((bm, 128), lambda i: (i, 0)),
        out_shape=jax.ShapeDtypeStruct((_N, 128), _F32),
    )(wvp, zp, S)


def _sc_edge(qh, kh, vh, ee, src, dst, zer):
    mesh = plsc.VectorSubcoreMesh(core_axis_name="c", subcore_axis_name="s")

    def _bufset():
        return [
            pltpu.VMEM((_C,), jnp.int32),
            pltpu.VMEM((_C,), jnp.int32),
            pltpu.VMEM((_C + 16,), jnp.int32),
            pltpu.VMEM((2 * _C,), jnp.int32),
            pltpu.VMEM((_C, 128), _F32),
            pltpu.VMEM((_C, 128), _F32),
            pltpu.VMEM((_C, 128), _F32),
            pltpu.VMEM((2 * _C, 128), _F32),
            pltpu.VMEM((_C * 8,), _F32),
            pltpu.SemaphoreType.DMA,
            pltpu.SemaphoreType.DMA,
            pltpu.SemaphoreType.DMA,
        ]

    @functools.partial(
        pl.kernel, mesh=mesh,
        out_type=[jax.ShapeDtypeStruct((2, _NAZ, 128), _F32),
                  jax.ShapeDtypeStruct((_E * 8,), _F32)],
        scratch_types=(_bufset() + _bufset()
                       + [pltpu.VMEM_SHARED((_NAZ, 128), _F32)]),
    )
    def k(qh_h, kh_h, vh_h, ee_h, src_h, dst_h, zer_h, wv_o, attn_o, *sc):
        bufs = (sc[0:12], sc[12:24])
        acc = sc[24]
        cid = lax.axis_index("c")
        sid = lax.axis_index("s")
        wid = sid * 2 + cid
        r0 = sid * _NT

        pltpu.sync_copy(zer_h.at[pl.ds(r0, _NT)], acc.at[pl.ds(r0, _NT)])
        plsc.subcore_barrier()

        lanes = lax.iota(jnp.int32, 16)
        zvec = jnp.zeros((16,), _F32)
        onev = jnp.ones((16,), _F32)
        hmasks_f = [jnp.where(lanes == hd, onev, zvec) for hd in range(_H)]
        xor_idx = [lanes ^ kk for kk in (8, 4, 2, 1)]
        rot8_idx = (lanes + 8) & 15
        gdn = lax.GatherDimensionNumbers(
            offset_dims=(), collapsed_slice_dims=(0,), start_index_map=(0,))

        def _shuf(x, idx):
            return lax.gather(x, idx[:, None], gdn, (1,),
                              mode=lax.GatherScatterMode.PROMISE_IN_BOUNDS)

        def _allsum(x):
            for xi in xor_idx:
                x = x + _shuf(x, xi)
            return x

        def base_of(j):
            return (wid + 32 * j) * _C

        def issue_idx(j, B):
            srcv, dstv, dstpv = bufs[B][0], bufs[B][1], bufs[B][2]
            isem = bufs[B][9]
            base = base_of(j)
            pltpu.async_copy(src_h.at[pl.ds(base, _C)], srcv, isem)
            pltpu.async_copy(dst_h.at[pl.ds(base, _C)], dstv, isem)
            pltpu.async_copy(dst_h.at[pl.ds(base, _C)],
                             dstpv.at[pl.ds(0, _C)], isem)

        def wait_idx(j, B):
            srcv, dstv, dstpv = bufs[B][0], bufs[B][1], bufs[B][2]
            isem = bufs[B][9]
            base = base_of(j)
            pltpu.make_async_copy(src_h.at[pl.ds(base, _C)], srcv, isem).wait()
            pltpu.make_async_copy(dst_h.at[pl.ds(base, _C)], dstv, isem).wait()
            pltpu.make_async_copy(dst_h.at[pl.ds(base, _C)],
                                  dstpv.at[pl.ds(0, _C)], isem).wait()

        def issue_gathers(j, B):
            srcv, dstv = bufs[B][0], bufs[B][1]
            kv, qv, ev, mzv = bufs[B][4:8]
            gsem = bufs[B][10]
            base = base_of(j)
            pltpu.async_copy(kh_h.at[srcv], kv, gsem)
            pltpu.async_copy(qh_h.at[dstv], qv, gsem)
            pltpu.async_copy(vh_h.at[srcv], mzv.at[pl.ds(0, _C)], gsem)
            pltpu.async_copy(ee_h.at[pl.ds(base, _C)], ev, gsem)

        def wait_gathers(j, B):
            srcv, dstv = bufs[B][0], bufs[B][1]
            kv, qv, ev, mzv = bufs[B][4:8]
            gsem = bufs[B][10]
            base = base_of(j)
            pltpu.make_async_copy(kh_h.at[srcv], kv, gsem).wait()
            pltpu.make_async_copy(qh_h.at[dstv], qv, gsem).wait()
            pltpu.make_async_copy(vh_h.at[srcv], mzv.at[pl.ds(0, _C)],
                                  gsem).wait()
            pltpu.make_async_copy(ee_h.at[pl.ds(base, _C)], ev, gsem).wait()

        def do_scatters(j, B):
            mzidxv, mzv, attnv = bufs[B][3], bufs[B][7], bufs[B][8]
            ssem = bufs[B][11]
            base = base_of(j)
            pltpu.sync_copy(mzv, acc.at[mzidxv], add=True)
            c = pltpu.async_copy(attnv, attn_o.at[pl.ds(base * 8, _C * 8)],
                                 ssem)
            c.wait()

        def compute(B):
            dstpv, mzidxv = bufs[B][2], bufs[B][3]
            kv, qv, ev, mzv, attnv = bufs[B][4:9]
            for o in (0, 16):
                d = dstpv[pl.ds(o, 16)]
                mzidxv[pl.ds(o, 16)] = d
                mzidxv[pl.ds(_C + o, 16)] = _ZB + lax.shift_right_logical(d, 4)

            def do_edge(ei):
                attn_vec = zvec
                for hd in range(_H):
                    sl = pl.ds(hd * 16, 16)
                    w = kv[ei, sl] * qv[ei, sl] * ev[ei, sl]
                    sv = jnp.exp(jnp.clip(_allsum(w), -5.0, 5.0))
                    mzv[ei, sl] = mzv[ei, sl] * sv
                    attn_vec = attn_vec + sv * hmasks_f[hd]
                d0 = dstpv[pl.ds(ei, 16)][0]
                off = (d0 & 15) * 8
                offc = jnp.minimum(off, 112)
                didx = (lanes - lax.broadcast_in_dim(off - offc, (16,), ())) & 15
                store_vec = _shuf(attn_vec, didx)
                for t in range(8):
                    mzv[_C + ei, pl.ds(t * 16, 16)] = zvec
                mzv[_C + ei, pl.ds(offc, 16)] = store_vec
                return attn_vec

            @plsc.parallel_loop(0, _C // 2, unroll=4)
            def _(e2):
                eA = e2 * 2
                aA = do_edge(eA)
                aB = do_edge(eA + 1)
                attnv[pl.ds(e2 * 16, 16)] = aA + _shuf(aB, rot8_idx)

        def pair_body(jp, carry):
            j0 = 2 * jp
            issue_idx(j0 + 1, 1)
            wait_gathers(j0, 0)
            compute(0)
            do_scatters(j0, 0)
            wait_idx(j0 + 1, 1)
            issue_gathers(j0 + 1, 1)

            j1 = j0 + 1

            @pl.when(jp < _NPAIR - 1)
            def _():
                issue_idx(j1 + 1, 0)
            wait_gathers(j1, 1)
            compute(1)
            do_scatters(j1, 1)

            @pl.when(jp < _NPAIR - 1)
            def _():
                wait_idx(j1 + 1, 0)
                issue_gathers(j1 + 1, 0)
            return carry

        issue_idx(0, 0)
        wait_idx(0, 0)
        issue_gathers(0, 0)
        lax.fori_loop(0, _NPAIR, pair_body, 0)

        @pl.when(wid < _TCH - 32 * _JFULL)
        def _():
            jt = _JFULL
            issue_idx(jt, 0)
            wait_idx(jt, 0)
            issue_gathers(jt, 0)
            wait_gathers(jt, 0)
            compute(0)
            do_scatters(jt, 0)

        plsc.subcore_barrier()
        pltpu.sync_copy(acc.at[pl.ds(r0, _NT)], wv_o.at[cid, pl.ds(r0, _NT)])

    return k(qh, kh, vh, ee, src, dst, zer)


def kernel(h, p, e, edge_index, Q_w, Q_b, K_w, K_b, E_w, E_b, V_w, V_b):
    W = jnp.concatenate([Q_w, K_w, V_w], axis=1)
    b = jnp.concatenate([Q_b, K_b, V_b])[None, :]
    qh, kh, vh = _qkv_call(h, p, W[:128], W[128:], b)
    ee = _proj_call(e, E_w * 0.25, E_b[None, :] * 0.25)
    src = edge_index[0]
    dst = edge_index[1]
    zer = jnp.zeros((_NAZ, 128), _F32)
    accd, attn_flat = _sc_edge(qh, kh, vh, ee, src, dst, zer)
    S = jnp.repeat(jnp.eye(_H, dtype=_F32), _D, axis=1)
    h_out = _combine_call(accd, accd.reshape(2, _NAZ * 16, 8), S)
    return (h_out.reshape(_N, _H, _D), attn_flat.reshape(_E, _H, 1))
